# Initial kernel scaffold; baseline (speedup 1.0000x reference)
import jax
import jax.numpy as jnp
from jax.experimental import pallas as pl


def kernel(x, shortcut, wt, bt, w1, scale1, shift1, w2, scale2, shift2, se_w1, se_w2):
    raise NotImplementedError("write your pallas kernel here")


# speedup vs baseline: 1.3017x; 1.3017x over previous
import functools

import jax
import jax.numpy as jnp
from jax.experimental import pallas as pl
from jax.experimental.pallas import tpu as pltpu


def _up_block_kernel(x_ref, sc_ref, wa_ref, bt2_ref, w1_ref, s1_ref, sh1_ref,
                     w2_ref, s2_ref, sh2_ref, se1_ref, se2_ref, o_ref,
                     pad_cat, pad_y, *, Hin, Win, Hs, Ws, Wp, Cup, Cs, Cout):
    f32 = jnp.float32
    Ccat = Cup + Cs

    pad_cat[...] = jnp.zeros((Hs + 2, Wp, Ccat), f32)
    pad_y[...] = jnp.zeros((Hs + 2, Wp, Cout), f32)

    xf = x_ref[0].reshape(Hin * Win, x_ref.shape[-1])
    vals = []
    for a in range(2):
        t = jnp.dot(xf, wa_ref[a], preferred_element_type=f32) + bt2_ref[...]
        vals.append(t.reshape(Hin, 1, 2 * Win, Cup))
    up = jnp.concatenate(vals, axis=1).reshape(Hs, Ws, Cup)
    pad_cat[1:Hs + 1, 1:Ws + 1, 0:Cup] = up
    pad_cat[1:Hs + 1, 1:Ws + 1, Cup:Ccat] = sc_ref[0].astype(f32)

    def conv3x3(pad_ref, w_ref, scale, shift, cin):
        acc = None
        for kh in range(3):
            rows = pad_ref[kh:kh + Hs, :, :].reshape(Hs * Wp, cin)
            d = jnp.dot(rows.astype(jnp.bfloat16), w_ref[kh],
                        preferred_element_type=f32)
            for kw in range(3):
                part = d[:, kw * Cout:(kw + 1) * Cout].reshape(Hs, Wp, Cout)
                contrib = part[:, kw:kw + Ws, :]
                acc = contrib if acc is None else acc + contrib
        return jnp.maximum(acc * scale + shift, 0.0)

    y1 = conv3x3(pad_cat, w1_ref, s1_ref[...], sh1_ref[...], Ccat)
    pad_y[1:Hs + 1, 1:Ws + 1, :] = y1

    y2 = conv3x3(pad_y, w2_ref, s2_ref[...], sh2_ref[...], Cout)

    pooled = jnp.mean(y2.reshape(Hs * Ws, Cout), axis=0, keepdims=True)
    h = jnp.maximum(
        jnp.dot(pooled, se1_ref[...], preferred_element_type=f32), 0.0)
    s = jax.nn.sigmoid(
        jnp.dot(h, se2_ref[...], preferred_element_type=f32))
    o_ref[...] = (y2 * s).reshape(1, Hs, Ws, Cout)


def kernel(x, shortcut, wt, bt, w1, scale1, shift1, w2, scale2, shift2,
           se_w1, se_w2):
    N, Hin, Win, Cin = x.shape
    Cup = wt.shape[1]
    Cs = shortcut.shape[-1]
    Ccat = Cup + Cs
    Cout = w1.shape[-1]
    Cr = se_w1.shape[-1]
    Hs, Ws = 2 * Hin, 2 * Win
    Wp = Ws + 8

    bf16 = jnp.bfloat16
    wa = jnp.transpose(wt, (2, 0, 3, 1)).reshape(2, Cin, 2 * Cup).astype(bf16)
    bt2 = jnp.tile(bt, 2).reshape(1, 2 * Cup)
    w1k = jnp.transpose(w1, (0, 2, 1, 3)).reshape(3, Ccat, 3 * Cout).astype(bf16)
    w2k = jnp.transpose(w2, (0, 2, 1, 3)).reshape(3, Cout, 3 * Cout).astype(bf16)

    kern = functools.partial(_up_block_kernel, Hin=Hin, Win=Win, Hs=Hs, Ws=Ws,
                             Wp=Wp, Cup=Cup, Cs=Cs, Cout=Cout)
    return pl.pallas_call(
        kern,
        out_shape=jax.ShapeDtypeStruct((N, Hs, Ws, Cout), jnp.float32),
        grid=(N,),
        in_specs=[
            pl.BlockSpec((1, Hin, Win, Cin), lambda n: (n, 0, 0, 0)),
            pl.BlockSpec((1, Hs, Ws, Cs), lambda n: (n, 0, 0, 0)),
            pl.BlockSpec((2, Cin, 2 * Cup), lambda n: (0, 0, 0)),
            pl.BlockSpec((1, 2 * Cup), lambda n: (0, 0)),
            pl.BlockSpec((3, Ccat, 3 * Cout), lambda n: (0, 0, 0)),
            pl.BlockSpec((1, Cout), lambda n: (0, 0)),
            pl.BlockSpec((1, Cout), lambda n: (0, 0)),
            pl.BlockSpec((3, Cout, 3 * Cout), lambda n: (0, 0, 0)),
            pl.BlockSpec((1, Cout), lambda n: (0, 0)),
            pl.BlockSpec((1, Cout), lambda n: (0, 0)),
            pl.BlockSpec((Cout, Cr), lambda n: (0, 0)),
            pl.BlockSpec((Cr, Cout), lambda n: (0, 0)),
        ],
        out_specs=pl.BlockSpec((1, Hs, Ws, Cout), lambda n: (n, 0, 0, 0)),
        scratch_shapes=[
            pltpu.VMEM((Hs + 2, Wp, Ccat), jnp.float32),
            pltpu.VMEM((Hs + 2, Wp, Cout), jnp.float32),
        ],
        compiler_params=pltpu.CompilerParams(dimension_semantics=("parallel",)),
    )(x.astype(bf16), shortcut.astype(bf16), wa, bt2, w1k, scale1.reshape(1, Cout),
      shift1.reshape(1, Cout), w2k, scale2.reshape(1, Cout),
      shift2.reshape(1, Cout), se_w1, se_w2)


# speedup vs baseline: 1.5554x; 1.1949x over previous
import functools

import jax
import jax.numpy as jnp
from jax.experimental import pallas as pl
from jax.experimental.pallas import tpu as pltpu


def _up_block_kernel(x_ref, sc_ref, wa_ref, bt2_ref, w1_ref, s1_ref, sh1_ref,
                     w2_ref, s2_ref, sh2_ref, se1_ref, se2_ref, o_ref,
                     pad_cat, pad_y, *, Hin, Win, Hs, Ws, Wp, Cup, Cs, Cout):
    f32 = jnp.float32
    bf16 = jnp.bfloat16
    Ccat = Cup + Cs

    for ref, c in ((pad_cat, Ccat), (pad_y, Cout)):
        ref[0:1, :, :] = jnp.zeros((1, Wp, c), f32)
        ref[Hs + 1:Hs + 2, :, :] = jnp.zeros((1, Wp, c), f32)
        ref[:, 0:1, :] = jnp.zeros((Hs + 2, 1, c), f32)
        ref[:, Ws + 1:Wp, :] = jnp.zeros((Hs + 2, Wp - Ws - 1, c), f32)

    xf = x_ref[0].reshape(Hin * Win, x_ref.shape[-1]).astype(bf16)
    vals = []
    for a in range(2):
        t = jnp.dot(xf, wa_ref[a], preferred_element_type=f32) + bt2_ref[...]
        vals.append(t.reshape(Hin, 1, 2 * Win, Cup))
    up = jnp.concatenate(vals, axis=1).reshape(Hs, Ws, Cup)
    pad_cat[1:Hs + 1, 1:Ws + 1, 0:Cup] = up
    pad_cat[1:Hs + 1, 1:Ws + 1, Cup:Ccat] = sc_ref[0]

    def conv3x3(pad_ref, w_ref, scale, shift, cin):
        acc = None
        for kh in range(3):
            rows = pad_ref[kh:kh + Hs, :, :].reshape(Hs * Wp, cin)
            d = jnp.dot(rows.astype(bf16), w_ref[kh],
                        preferred_element_type=f32)
            for kw in range(3):
                part = d[:, kw * Cout:(kw + 1) * Cout].reshape(Hs, Wp, Cout)
                contrib = part[:, kw:kw + Ws, :]
                acc = contrib if acc is None else acc + contrib
        return jnp.maximum(acc * scale + shift, 0.0)

    y1 = conv3x3(pad_cat, w1_ref, s1_ref[...], sh1_ref[...], Ccat)
    pad_y[1:Hs + 1, 1:Ws + 1, :] = y1

    y2 = conv3x3(pad_y, w2_ref, s2_ref[...], sh2_ref[...], Cout)

    pooled = jnp.mean(y2.reshape(Hs * Ws, Cout), axis=0, keepdims=True)
    h = jnp.maximum(
        jnp.dot(pooled, se1_ref[...], preferred_element_type=f32), 0.0)
    s = jax.nn.sigmoid(
        jnp.dot(h, se2_ref[...], preferred_element_type=f32))
    o_ref[...] = (y2 * s).reshape(1, Hs, Ws, Cout)


def kernel(x, shortcut, wt, bt, w1, scale1, shift1, w2, scale2, shift2,
           se_w1, se_w2):
    N, Hin, Win, Cin = x.shape
    Cup = wt.shape[1]
    Cs = shortcut.shape[-1]
    Ccat = Cup + Cs
    Cout = w1.shape[-1]
    Cr = se_w1.shape[-1]
    Hs, Ws = 2 * Hin, 2 * Win
    Wp = Ws + 8

    bf16 = jnp.bfloat16
    wa = jnp.transpose(wt, (2, 0, 3, 1)).reshape(2, Cin, 2 * Cup).astype(bf16)
    bt2 = jnp.tile(bt, 2).reshape(1, 2 * Cup)
    w1k = jnp.transpose(w1, (0, 2, 1, 3)).reshape(3, Ccat, 3 * Cout).astype(bf16)
    w2k = jnp.transpose(w2, (0, 2, 1, 3)).reshape(3, Cout, 3 * Cout).astype(bf16)

    kern = functools.partial(_up_block_kernel, Hin=Hin, Win=Win, Hs=Hs, Ws=Ws,
                             Wp=Wp, Cup=Cup, Cs=Cs, Cout=Cout)
    return pl.pallas_call(
        kern,
        out_shape=jax.ShapeDtypeStruct((N, Hs, Ws, Cout), jnp.float32),
        grid=(N,),
        in_specs=[
            pl.BlockSpec((1, Hin, Win, Cin), lambda n: (n, 0, 0, 0)),
            pl.BlockSpec((1, Hs, Ws, Cs), lambda n: (n, 0, 0, 0)),
            pl.BlockSpec((2, Cin, 2 * Cup), lambda n: (0, 0, 0)),
            pl.BlockSpec((1, 2 * Cup), lambda n: (0, 0)),
            pl.BlockSpec((3, Ccat, 3 * Cout), lambda n: (0, 0, 0)),
            pl.BlockSpec((1, Cout), lambda n: (0, 0)),
            pl.BlockSpec((1, Cout), lambda n: (0, 0)),
            pl.BlockSpec((3, Cout, 3 * Cout), lambda n: (0, 0, 0)),
            pl.BlockSpec((1, Cout), lambda n: (0, 0)),
            pl.BlockSpec((1, Cout), lambda n: (0, 0)),
            pl.BlockSpec((Cout, Cr), lambda n: (0, 0)),
            pl.BlockSpec((Cr, Cout), lambda n: (0, 0)),
        ],
        out_specs=pl.BlockSpec((1, Hs, Ws, Cout), lambda n: (n, 0, 0, 0)),
        scratch_shapes=[
            pltpu.VMEM((Hs + 2, Wp, Ccat), jnp.float32),
            pltpu.VMEM((Hs + 2, Wp, Cout), jnp.float32),
        ],
        compiler_params=pltpu.CompilerParams(dimension_semantics=("parallel",)),
    )(x, shortcut, wa, bt2, w1k, scale1.reshape(1, Cout),
      shift1.reshape(1, Cout), w2k, scale2.reshape(1, Cout),
      shift2.reshape(1, Cout), se_w1, se_w2)


# speedup vs baseline: 1.6093x; 1.0347x over previous
import functools

import jax
import jax.numpy as jnp
from jax.experimental import pallas as pl
from jax.experimental.pallas import tpu as pltpu


def _up_block_kernel(x_ref, sc_ref, wa_ref, bt2_ref, w1_ref, s1_ref, sh1_ref,
                     w2_ref, s2_ref, sh2_ref, se1_ref, se2_ref, o_ref,
                     pad_cat, pad_y, *, Hin, Win, Hs, Ws, Wp, Cup, Cs, Cout):
    f32 = jnp.float32
    bf16 = jnp.bfloat16
    Ccat = Cup + Cs

    for ref, c in ((pad_cat, Ccat), (pad_y, Cout)):
        ref[0:1, :, :] = jnp.zeros((1, Wp, c), f32)
        ref[Hs + 1:Hs + 2, :, :] = jnp.zeros((1, Wp, c), f32)
        ref[:, 0:1, :] = jnp.zeros((Hs + 2, 1, c), f32)
        ref[:, Ws + 1:Wp, :] = jnp.zeros((Hs + 2, Wp - Ws - 1, c), f32)

    xf = x_ref[0].reshape(Hin * Win, x_ref.shape[-1]).astype(bf16)
    vals = []
    for a in range(2):
        t = jnp.dot(xf, wa_ref[a], preferred_element_type=f32) + bt2_ref[...]
        vals.append(t.reshape(Hin, 1, 2 * Win, Cup))
    up = jnp.concatenate(vals, axis=1).reshape(Hs, Ws, Cup)
    pad_cat[1:Hs + 1, 1:Ws + 1, 0:Cup] = up
    pad_cat[1:Hs + 1, 1:Ws + 1, Cup:Ccat] = sc_ref[0]

    Ch = 8 if Hs % 8 == 0 else Hs

    def conv3x3_chunk(pad_ref, w_ref, scale, shift, cin, h0):
        acc = None
        for kh in range(3):
            rows = pad_ref[h0 + kh:h0 + kh + Ch, :, :].reshape(Ch * Wp, cin)
            d = jnp.dot(rows.astype(bf16), w_ref[kh],
                        preferred_element_type=f32)
            for kw in range(3):
                part = d[:, kw * Cout:(kw + 1) * Cout].reshape(Ch, Wp, Cout)
                contrib = part[:, kw:kw + Ws, :]
                acc = contrib if acc is None else acc + contrib
        return jnp.maximum(acc * scale + shift, 0.0)

    for h0 in range(0, Hs, Ch):
        y1 = conv3x3_chunk(pad_cat, w1_ref, s1_ref[...], sh1_ref[...], Ccat, h0)
        pad_y[h0 + 1:h0 + Ch + 1, 1:Ws + 1, :] = y1

    pooled = jnp.zeros((1, Cout), f32)
    for h0 in range(0, Hs, Ch):
        y2 = conv3x3_chunk(pad_y, w2_ref, s2_ref[...], sh2_ref[...], Cout, h0)
        pooled = pooled + jnp.sum(y2.reshape(Ch * Ws, Cout), axis=0,
                                  keepdims=True)
        o_ref[0, h0:h0 + Ch, :, :] = y2

    pooled = pooled * (1.0 / (Hs * Ws))
    h = jnp.maximum(
        jnp.dot(pooled, se1_ref[...], preferred_element_type=f32), 0.0)
    s = jax.nn.sigmoid(
        jnp.dot(h, se2_ref[...], preferred_element_type=f32))
    o_ref[...] = o_ref[...] * s


def kernel(x, shortcut, wt, bt, w1, scale1, shift1, w2, scale2, shift2,
           se_w1, se_w2):
    N, Hin, Win, Cin = x.shape
    Cup = wt.shape[1]
    Cs = shortcut.shape[-1]
    Ccat = Cup + Cs
    Cout = w1.shape[-1]
    Cr = se_w1.shape[-1]
    Hs, Ws = 2 * Hin, 2 * Win
    Wp = Ws + 8

    bf16 = jnp.bfloat16
    wa = jnp.transpose(wt, (2, 0, 3, 1)).reshape(2, Cin, 2 * Cup).astype(bf16)
    bt2 = jnp.tile(bt, 2).reshape(1, 2 * Cup)
    w1k = jnp.transpose(w1, (0, 2, 1, 3)).reshape(3, Ccat, 3 * Cout).astype(bf16)
    w2k = jnp.transpose(w2, (0, 2, 1, 3)).reshape(3, Cout, 3 * Cout).astype(bf16)

    kern = functools.partial(_up_block_kernel, Hin=Hin, Win=Win, Hs=Hs, Ws=Ws,
                             Wp=Wp, Cup=Cup, Cs=Cs, Cout=Cout)
    return pl.pallas_call(
        kern,
        out_shape=jax.ShapeDtypeStruct((N, Hs, Ws, Cout), jnp.float32),
        grid=(N,),
        in_specs=[
            pl.BlockSpec((1, Hin, Win, Cin), lambda n: (n, 0, 0, 0)),
            pl.BlockSpec((1, Hs, Ws, Cs), lambda n: (n, 0, 0, 0)),
            pl.BlockSpec((2, Cin, 2 * Cup), lambda n: (0, 0, 0)),
            pl.BlockSpec((1, 2 * Cup), lambda n: (0, 0)),
            pl.BlockSpec((3, Ccat, 3 * Cout), lambda n: (0, 0, 0)),
            pl.BlockSpec((1, Cout), lambda n: (0, 0)),
            pl.BlockSpec((1, Cout), lambda n: (0, 0)),
            pl.BlockSpec((3, Cout, 3 * Cout), lambda n: (0, 0, 0)),
            pl.BlockSpec((1, Cout), lambda n: (0, 0)),
            pl.BlockSpec((1, Cout), lambda n: (0, 0)),
            pl.BlockSpec((Cout, Cr), lambda n: (0, 0)),
            pl.BlockSpec((Cr, Cout), lambda n: (0, 0)),
        ],
        out_specs=pl.BlockSpec((1, Hs, Ws, Cout), lambda n: (n, 0, 0, 0)),
        scratch_shapes=[
            pltpu.VMEM((Hs + 2, Wp, Ccat), jnp.float32),
            pltpu.VMEM((Hs + 2, Wp, Cout), jnp.float32),
        ],
        compiler_params=pltpu.CompilerParams(dimension_semantics=("parallel",)),
    )(x, shortcut, wa, bt2, w1k, scale1.reshape(1, Cout),
      shift1.reshape(1, Cout), w2k, scale2.reshape(1, Cout),
      shift2.reshape(1, Cout), se_w1, se_w2)


# speedup vs baseline: 1.6128x; 1.0022x over previous
import functools

import jax
import jax.numpy as jnp
from jax.experimental import pallas as pl
from jax.experimental.pallas import tpu as pltpu


def _up_block_kernel(x_ref, sc_ref, wa_ref, bt2_ref, w1_ref, s1_ref, sh1_ref,
                     w2_ref, s2_ref, sh2_ref, se1_ref, se2_ref, o_ref,
                     pad_cat, pad_y, *, Hin, Win, Hs, Ws, Wp, Cup, Cs, Cout):
    f32 = jnp.float32
    bf16 = jnp.bfloat16
    Ccat = Cup + Cs

    for ref, c in ((pad_cat, Ccat), (pad_y, Cout)):
        ref[0:1, :, :] = jnp.zeros((1, Wp, c), f32)
        ref[Hs + 1:Hs + 2, :, :] = jnp.zeros((1, Wp, c), f32)
        ref[:, 0:1, :] = jnp.zeros((Hs + 2, 1, c), f32)
        ref[:, Ws + 1:Wp, :] = jnp.zeros((Hs + 2, Wp - Ws - 1, c), f32)

    Ch = 8 if Hs % 8 == 0 else Hs

    def conv3x3_chunk(pad_ref, w_ref, scale, shift, cin, h0):
        acc = None
        for kh in range(3):
            rows = pad_ref[h0 + kh:h0 + kh + Ch, :, :].reshape(Ch * Wp, cin)
            d = jnp.dot(rows.astype(bf16), w_ref[kh],
                        preferred_element_type=f32)
            for kw in range(3):
                part = d[:, kw * Cout:(kw + 1) * Cout].reshape(Ch, Wp, Cout)
                contrib = part[:, kw:kw + Ws, :]
                acc = contrib if acc is None else acc + contrib
        return jnp.maximum(acc * scale + shift, 0.0)

    for img in range(x_ref.shape[0]):
        xf = x_ref[img].reshape(Hin * Win, x_ref.shape[-1]).astype(bf16)
        vals = []
        for a in range(2):
            t = jnp.dot(xf, wa_ref[a], preferred_element_type=f32) + bt2_ref[...]
            vals.append(t.reshape(Hin, 1, 2 * Win, Cup))
        up = jnp.concatenate(vals, axis=1).reshape(Hs, Ws, Cup)
        pad_cat[1:Hs + 1, 1:Ws + 1, 0:Cup] = up
        pad_cat[1:Hs + 1, 1:Ws + 1, Cup:Ccat] = sc_ref[img]

        for h0 in range(0, Hs, Ch):
            y1 = conv3x3_chunk(pad_cat, w1_ref, s1_ref[...], sh1_ref[...],
                               Ccat, h0)
            pad_y[h0 + 1:h0 + Ch + 1, 1:Ws + 1, :] = y1

        pooled = jnp.zeros((1, Cout), f32)
        for h0 in range(0, Hs, Ch):
            y2 = conv3x3_chunk(pad_y, w2_ref, s2_ref[...], sh2_ref[...],
                               Cout, h0)
            pooled = pooled + jnp.sum(y2.reshape(Ch * Ws, Cout), axis=0,
                                      keepdims=True)
            o_ref[img, h0:h0 + Ch, :, :] = y2

        pooled = pooled * (1.0 / (Hs * Ws))
        h = jnp.maximum(
            jnp.dot(pooled, se1_ref[...], preferred_element_type=f32), 0.0)
        s = jax.nn.sigmoid(
            jnp.dot(h, se2_ref[...], preferred_element_type=f32))
        o_ref[img, :, :, :] = o_ref[img, :, :, :] * s


def kernel(x, shortcut, wt, bt, w1, scale1, shift1, w2, scale2, shift2,
           se_w1, se_w2):
    N, Hin, Win, Cin = x.shape
    Cup = wt.shape[1]
    Cs = shortcut.shape[-1]
    Ccat = Cup + Cs
    Cout = w1.shape[-1]
    Cr = se_w1.shape[-1]
    Hs, Ws = 2 * Hin, 2 * Win
    Wp = Ws + 8
    IB = 2 if N % 2 == 0 else 1

    bf16 = jnp.bfloat16
    wa = jnp.transpose(wt, (2, 0, 3, 1)).reshape(2, Cin, 2 * Cup).astype(bf16)
    bt2 = jnp.tile(bt, 2).reshape(1, 2 * Cup)
    w1k = jnp.transpose(w1, (0, 2, 1, 3)).reshape(3, Ccat, 3 * Cout).astype(bf16)
    w2k = jnp.transpose(w2, (0, 2, 1, 3)).reshape(3, Cout, 3 * Cout).astype(bf16)

    kern = functools.partial(_up_block_kernel, Hin=Hin, Win=Win, Hs=Hs, Ws=Ws,
                             Wp=Wp, Cup=Cup, Cs=Cs, Cout=Cout)
    return pl.pallas_call(
        kern,
        out_shape=jax.ShapeDtypeStruct((N, Hs, Ws, Cout), jnp.float32),
        grid=(N // IB,),
        in_specs=[
            pl.BlockSpec((IB, Hin, Win, Cin), lambda n: (n, 0, 0, 0)),
            pl.BlockSpec((IB, Hs, Ws, Cs), lambda n: (n, 0, 0, 0)),
            pl.BlockSpec((2, Cin, 2 * Cup), lambda n: (0, 0, 0)),
            pl.BlockSpec((1, 2 * Cup), lambda n: (0, 0)),
            pl.BlockSpec((3, Ccat, 3 * Cout), lambda n: (0, 0, 0)),
            pl.BlockSpec((1, Cout), lambda n: (0, 0)),
            pl.BlockSpec((1, Cout), lambda n: (0, 0)),
            pl.BlockSpec((3, Cout, 3 * Cout), lambda n: (0, 0, 0)),
            pl.BlockSpec((1, Cout), lambda n: (0, 0)),
            pl.BlockSpec((1, Cout), lambda n: (0, 0)),
            pl.BlockSpec((Cout, Cr), lambda n: (0, 0)),
            pl.BlockSpec((Cr, Cout), lambda n: (0, 0)),
        ],
        out_specs=pl.BlockSpec((IB, Hs, Ws, Cout), lambda n: (n, 0, 0, 0)),
        scratch_shapes=[
            pltpu.VMEM((Hs + 2, Wp, Ccat), jnp.float32),
            pltpu.VMEM((Hs + 2, Wp, Cout), jnp.float32),
        ],
        compiler_params=pltpu.CompilerParams(dimension_semantics=("parallel",)),
    )(x, shortcut, wa, bt2, w1k, scale1.reshape(1, Cout),
      shift1.reshape(1, Cout), w2k, scale2.reshape(1, Cout),
      shift2.reshape(1, Cout), se_w1, se_w2)


# speedup vs baseline: 1.6804x; 1.0419x over previous
import functools

import jax
import jax.numpy as jnp
from jax.experimental import pallas as pl
from jax.experimental.pallas import tpu as pltpu


def _up_block_kernel(x_ref, sc_ref, wa_ref, bt_ref, w1_ref, s1_ref, sh1_ref,
                     w2_ref, s2_ref, sh2_ref, se1_ref, se2_ref, o_ref,
                     pad_cat, pad_y, w1b, w2b, *, Hin, Win, Hs, Ws, Wp, Cup,
                     Cs, Cout):
    f32 = jnp.float32
    bf16 = jnp.bfloat16
    Ccat = Cup + Cs

    @pl.when(pl.program_id(0) == 0)
    def _init():
        for ref, c in ((pad_cat, Ccat), (pad_y, Cout)):
            ref[0:1, :, :] = jnp.zeros((1, Wp, c), f32)
            ref[Hs + 1:Hs + 2, :, :] = jnp.zeros((1, Wp, c), f32)
            ref[:, 0:1, :] = jnp.zeros((Hs + 2, 1, c), f32)
            ref[:, Ws + 1:Wp, :] = jnp.zeros((Hs + 2, Wp - Ws - 1, c), f32)
        for kh in range(3):
            for kw in range(3):
                k = kh * 3 + kw
                w1b[kh, :, kw * Cout:(kw + 1) * Cout] = (
                    w1_ref[k * Ccat:(k + 1) * Ccat, :].astype(bf16))
                w2b[kh, :, kw * Cout:(kw + 1) * Cout] = (
                    w2_ref[k * Cout:(k + 1) * Cout, :].astype(bf16))

    bt2 = jnp.concatenate([bt_ref[...], bt_ref[...]], axis=-1)

    Ch = 8 if Hs % 8 == 0 else Hs

    def conv3x3_chunk(pad_ref, w_ref, scale, shift, cin, h0):
        acc = None
        for kh in range(3):
            rows = pad_ref[h0 + kh:h0 + kh + Ch, :, :].reshape(Ch * Wp, cin)
            d = jnp.dot(rows.astype(bf16), w_ref[kh],
                        preferred_element_type=f32)
            for kw in range(3):
                part = d[:, kw * Cout:(kw + 1) * Cout].reshape(Ch, Wp, Cout)
                contrib = part[:, kw:kw + Ws, :]
                acc = contrib if acc is None else acc + contrib
        return jnp.maximum(acc * scale + shift, 0.0)

    for img in range(x_ref.shape[0]):
        xf = x_ref[img].reshape(Hin * Win, x_ref.shape[-1]).astype(bf16)
        vals = []
        for a in range(2):
            t = jnp.dot(xf, wa_ref[a], preferred_element_type=f32) + bt2
            vals.append(t.reshape(Hin, 1, 2 * Win, Cup))
        up = jnp.concatenate(vals, axis=1).reshape(Hs, Ws, Cup)
        pad_cat[1:Hs + 1, 1:Ws + 1, 0:Cup] = up
        pad_cat[1:Hs + 1, 1:Ws + 1, Cup:Ccat] = sc_ref[img]

        for h0 in range(0, Hs, Ch):
            y1 = conv3x3_chunk(pad_cat, w1b, s1_ref[...], sh1_ref[...],
                               Ccat, h0)
            pad_y[h0 + 1:h0 + Ch + 1, 1:Ws + 1, :] = y1

        pooled = jnp.zeros((1, Cout), f32)
        for h0 in range(0, Hs, Ch):
            y2 = conv3x3_chunk(pad_y, w2b, s2_ref[...], sh2_ref[...],
                               Cout, h0)
            pooled = pooled + jnp.sum(y2.reshape(Ch * Ws, Cout), axis=0,
                                      keepdims=True)
            o_ref[img, h0:h0 + Ch, :, :] = y2

        pooled = pooled * (1.0 / (Hs * Ws))
        h = jnp.maximum(
            jnp.dot(pooled, se1_ref[...], preferred_element_type=f32), 0.0)
        s = jax.nn.sigmoid(
            jnp.dot(h, se2_ref[...], preferred_element_type=f32))
        o_ref[img, :, :, :] = o_ref[img, :, :, :] * s


def kernel(x, shortcut, wt, bt, w1, scale1, shift1, w2, scale2, shift2,
           se_w1, se_w2):
    N, Hin, Win, Cin = x.shape
    Cup = wt.shape[1]
    Cs = shortcut.shape[-1]
    Ccat = Cup + Cs
    Cout = w1.shape[-1]
    Cr = se_w1.shape[-1]
    Hs, Ws = 2 * Hin, 2 * Win
    Wp = Ws + 8
    IB = 2 if N % 2 == 0 else 1

    bf16 = jnp.bfloat16
    wa = jnp.transpose(wt, (2, 0, 3, 1)).reshape(2, Cin, 2 * Cup).astype(bf16)
    w1k = w1.reshape(9 * Ccat, Cout)
    w2k = w2.reshape(9 * Cout, Cout)

    kern = functools.partial(_up_block_kernel, Hin=Hin, Win=Win, Hs=Hs, Ws=Ws,
                             Wp=Wp, Cup=Cup, Cs=Cs, Cout=Cout)
    return pl.pallas_call(
        kern,
        out_shape=jax.ShapeDtypeStruct((N, Hs, Ws, Cout), jnp.float32),
        grid=(N // IB,),
        in_specs=[
            pl.BlockSpec((IB, Hin, Win, Cin), lambda n: (n, 0, 0, 0)),
            pl.BlockSpec((IB, Hs, Ws, Cs), lambda n: (n, 0, 0, 0)),
            pl.BlockSpec((2, Cin, 2 * Cup), lambda n: (0, 0, 0)),
            pl.BlockSpec((1, Cup), lambda n: (0, 0)),
            pl.BlockSpec((9 * Ccat, Cout), lambda n: (0, 0)),
            pl.BlockSpec((1, Cout), lambda n: (0, 0)),
            pl.BlockSpec((1, Cout), lambda n: (0, 0)),
            pl.BlockSpec((9 * Cout, Cout), lambda n: (0, 0)),
            pl.BlockSpec((1, Cout), lambda n: (0, 0)),
            pl.BlockSpec((1, Cout), lambda n: (0, 0)),
            pl.BlockSpec((Cout, Cr), lambda n: (0, 0)),
            pl.BlockSpec((Cr, Cout), lambda n: (0, 0)),
        ],
        out_specs=pl.BlockSpec((IB, Hs, Ws, Cout), lambda n: (n, 0, 0, 0)),
        scratch_shapes=[
            pltpu.VMEM((Hs + 2, Wp, Ccat), jnp.float32),
            pltpu.VMEM((Hs + 2, Wp, Cout), jnp.float32),
            pltpu.VMEM((3, Ccat, 3 * Cout), jnp.bfloat16),
            pltpu.VMEM((3, Cout, 3 * Cout), jnp.bfloat16),
        ],
        compiler_params=pltpu.CompilerParams(dimension_semantics=("parallel",)),
    )(x, shortcut, wa, bt.reshape(1, Cup), w1k, scale1.reshape(1, Cout),
      shift1.reshape(1, Cout), w2k, scale2.reshape(1, Cout),
      shift2.reshape(1, Cout), se_w1, se_w2)
